# baseline (speedup 1.0000x reference)
import functools

import jax
import jax.numpy as jnp
from jax.experimental import pallas as pl
from jax.experimental.pallas import tpu as pltpu

N = 10000
E = 160000
IN = 128
HID = 512
HEADS = 2
OUT = 10


def _mlp_body(g_ref, w1_ref, b1_ref, w2_ref, b2_ref, w3_ref, b3_ref, o_ref):
    g1 = jnp.maximum(
        jnp.dot(g_ref[...], w1_ref[...], preferred_element_type=jnp.float32)
        + b1_ref[...], 0.0)
    g2 = jnp.maximum(
        jnp.dot(g1, w2_ref[...], preferred_element_type=jnp.float32)
        + b2_ref[...], 0.0)
    o_ref[...] = (
        jnp.dot(g2, w3_ref[...], preferred_element_type=jnp.float32)
        + b3_ref[...])


def _mlp_head(g, Wf1, bf1, Wf2, bf2, Wf3, bf3):
    M = g.shape[0]
    MP = 1280
    gp = jnp.zeros((MP, 4096), jnp.float32).at[:M].set(g)
    w3p = jnp.zeros((128, 128), jnp.float32).at[:, :OUT].set(Wf3)
    b3p = jnp.zeros((128,), jnp.float32).at[:OUT].set(bf3)
    out = pl.pallas_call(
        _mlp_body,
        out_shape=jax.ShapeDtypeStruct((MP, 128), jnp.float32),
    )(gp, Wf1, bf1.reshape(1, -1), Wf2, bf2.reshape(1, -1), w3p,
      b3p.reshape(1, -1))
    return out[:M, :OUT]


def _gat_layer(x, edge_index, W, a_src, a_dst, b, heads, C):
    n = x.shape[0]
    loop = jnp.arange(n, dtype=edge_index.dtype)
    src = jnp.concatenate([edge_index[0], loop])
    dst = jnp.concatenate([edge_index[1], loop])
    h = (x @ W).reshape(n, heads, C)
    alpha_src = jnp.sum(h * a_src, axis=-1)
    alpha_dst = jnp.sum(h * a_dst, axis=-1)
    e = alpha_src[src] + alpha_dst[dst]
    e = jnp.where(e > 0, e, 0.2 * e)
    m = jax.ops.segment_max(e, dst, num_segments=n)
    m = jax.lax.stop_gradient(jnp.where(jnp.isfinite(m), m, 0.0))
    ex = jnp.exp(e - m[dst])
    den = jax.ops.segment_sum(ex, dst, num_segments=n)
    att = ex / (den[dst] + 1e-16)
    msg = h[src] * att[:, :, None]
    out = jax.ops.segment_sum(msg, dst, num_segments=n)
    return out.reshape(n, heads * C) + b


def kernel(x, edge_index, W1, att_src1, att_dst1, b1, W2, att_src2,
           att_dst2, b2, Wf1, bf1, Wf2, bf2, Wf3, bf3):
    h1 = jax.nn.relu(_gat_layer(x, edge_index, W1, att_src1, att_dst1, b1,
                                HEADS, HID))
    h2 = jax.nn.relu(_gat_layer(h1, edge_index, W2, att_src2, att_dst2, b2,
                                1, HID))
    g = h2.reshape(-1, 8 * HID)
    return _mlp_head(g, Wf1, bf1, Wf2, bf2, Wf3, bf3)
